# baseline (speedup 1.0000x reference)
import jax
import jax.numpy as jnp
from jax import lax
from jax.experimental import pallas as pl
from jax.experimental.pallas import tpu as pltpu
from jax.experimental.pallas import tpu_sc as plsc

U = 50000
I = 50000
D = 128
B = 4096
DEG = 32

NC = 2
NS = 16
NW = NC * NS
BW = B // NW


def _transpose_adj(adj, adjT):
    lanes = lax.iota(jnp.int32, 16)

    def jbody(j, c):
        jv = jnp.full((16,), j, jnp.int32)
        for g in range(BW // 16):
            rows = lanes + (g * 16)
            vals = plsc.load_gather(adj, [rows, jv])
            adjT[j, pl.ds(g * 16, 16)] = vals
        return c

    lax.fori_loop(0, DEG, jbody, 0)


def _fire_mean_gather(table, adjT, acc, sem):
    pltpu.async_copy(table.at[adjT.at[0]], acc, sem).wait()

    def jbody(j, c):
        pltpu.async_copy(table.at[adjT.at[j]], acc, sem, add=True)
        return c

    lax.fori_loop(1, DEG, jbody, 0)


def _drain(table, acc, sem, count):
    def dbody(j, c):
        pltpu.make_async_copy(table.at[pl.ds(0, BW)], acc, sem).wait()
        return c

    lax.fori_loop(0, count, dbody, 0)


def _scale(accA, c0):
    def ibody(i, c):
        for g in range(D // 16):
            s = pl.ds(g * 16, 16)
            accA[i, s] = accA[i, s] * c0
        return c

    lax.fori_loop(0, BW, ibody, 0)


def _sc_a_body(v2e, nodes_u, nodes_v, u_v, coef,
               hist_o, self_v_o,
               nodes_b, nodes_b2, adj, adjT, accA, accB, coef_v, sem, sem2):
    wid = lax.axis_index("s") * NC + lax.axis_index("c")
    base = wid * BW

    pltpu.sync_copy(coef, coef_v)
    c_inv = coef_v[pl.ds(0, 16)]

    pltpu.sync_copy(nodes_u.at[pl.ds(base, BW)], nodes_b)
    adj_cp = pltpu.async_copy(u_v.at[nodes_b], adj, sem2)
    pltpu.sync_copy(nodes_v.at[pl.ds(base, BW)], nodes_b2)
    sv = pltpu.async_copy(v2e.at[nodes_b2], accB, sem)

    adj_cp.wait()
    _transpose_adj(adj, adjT)
    sv.wait()
    pltpu.sync_copy(accB, self_v_o.at[pl.ds(base, BW)])

    _fire_mean_gather(v2e, adjT, accA, sem)
    _drain(v2e, accA, sem, DEG - 1)
    _scale(accA, c_inv)
    pltpu.sync_copy(accA, hist_o.at[pl.ds(base, BW)])


def _sc_b_body(mixed, nodes_u, nodes_v, u_u, v_u, coef,
               self_u_o, soc_o, vagg_o,
               nodes_b, nodes_b2, adj, adj2, adjT, adjT2, accA, accB,
               coef_v, sem, sem2, sem3):
    wid = lax.axis_index("s") * NC + lax.axis_index("c")
    base = wid * BW

    pltpu.sync_copy(coef, coef_v)
    c_inv = coef_v[pl.ds(0, 16)]

    pltpu.sync_copy(nodes_u.at[pl.ds(base, BW)], nodes_b)
    adj_cp = pltpu.async_copy(u_u.at[nodes_b], adj, sem2)
    su = pltpu.async_copy(mixed.at[nodes_b], accB, sem)
    pltpu.sync_copy(nodes_v.at[pl.ds(base, BW)], nodes_b2)
    adj_cp2 = pltpu.async_copy(v_u.at[nodes_b2], adj2, sem2)

    adj_cp.wait()
    _transpose_adj(adj, adjT)
    su.wait()
    pltpu.sync_copy(accB, self_u_o.at[pl.ds(base, BW)])

    _fire_mean_gather(mixed, adjT, accA, sem)
    adj_cp2.wait()
    _transpose_adj(adj2, adjT2)
    _fire_mean_gather(mixed, adjT2, accB, sem3)
    _drain(mixed, accA, sem, DEG - 1)
    _scale(accA, c_inv)
    pltpu.sync_copy(accA, soc_o.at[pl.ds(base, BW)])
    _drain(mixed, accB, sem3, DEG - 1)
    _scale(accB, c_inv)
    pltpu.sync_copy(accB, vagg_o.at[pl.ds(base, BW)])


def _sc_mesh():
    return plsc.VectorSubcoreMesh(
        core_axis_name="c", subcore_axis_name="s",
        num_cores=NC, num_subcores=NS)


def _sc_a(v2e, nodes_u, nodes_v, u_vp, coef):
    f32 = jnp.float32
    i32 = jnp.int32
    run = pl.kernel(
        _sc_a_body,
        out_type=tuple(jax.ShapeDtypeStruct((B, D), f32) for _ in range(2)),
        mesh=_sc_mesh(),
        compiler_params=pltpu.CompilerParams(needs_layout_passes=False),
        scratch_types=[
            pltpu.VMEM((BW,), i32),
            pltpu.VMEM((BW,), i32),
            pltpu.VMEM((BW, 128), i32),
            pltpu.VMEM((DEG, BW), i32),
            pltpu.VMEM((BW, D), f32),
            pltpu.VMEM((BW, D), f32),
            pltpu.VMEM((128,), f32),
            pltpu.SemaphoreType.DMA,
            pltpu.SemaphoreType.DMA,
        ],
    )
    return run(v2e, nodes_u, nodes_v, u_vp, coef)


def _sc_b(mixed, nodes_u, nodes_v, u_up, v_up, coef):
    f32 = jnp.float32
    i32 = jnp.int32
    run = pl.kernel(
        _sc_b_body,
        out_type=tuple(jax.ShapeDtypeStruct((B, D), f32) for _ in range(3)),
        mesh=_sc_mesh(),
        compiler_params=pltpu.CompilerParams(needs_layout_passes=False),
        scratch_types=[
            pltpu.VMEM((BW,), i32),
            pltpu.VMEM((BW,), i32),
            pltpu.VMEM((BW, 128), i32),
            pltpu.VMEM((BW, 128), i32),
            pltpu.VMEM((DEG, BW), i32),
            pltpu.VMEM((DEG, BW), i32),
            pltpu.VMEM((BW, D), f32),
            pltpu.VMEM((BW, D), f32),
            pltpu.VMEM((128,), f32),
            pltpu.SemaphoreType.DMA,
            pltpu.SemaphoreType.DMA,
            pltpu.SemaphoreType.DMA,
        ],
    )
    return run(mixed, nodes_u, nodes_v, u_up, v_up, coef)


_MIXR = 4096
_PADR = 4096


def _mix_body(r_ref, t_ref, c_ref, o_ref):
    bf16 = jnp.bfloat16
    f32 = jnp.float32
    r = r_ref[...].astype(bf16).astype(f32)
    t = t_ref[...].astype(bf16).astype(f32)
    o_ref[...] = c_ref[0, 0] * r + c_ref[0, 1] * t


def _mix_table(u2e_r, u2e_t, csc):
    f32 = jnp.float32
    n = u2e_r.shape[0]
    grid = (n + _MIXR - 1) // _MIXR
    return pl.pallas_call(
        _mix_body,
        grid=(grid,),
        in_specs=[
            pl.BlockSpec((_MIXR, D), lambda i: (i, 0)),
            pl.BlockSpec((_MIXR, D), lambda i: (i, 0)),
            pl.BlockSpec(memory_space=pltpu.SMEM),
        ],
        out_specs=pl.BlockSpec((_MIXR, D), lambda i: (i, 0)),
        out_shape=jax.ShapeDtypeStruct((n, D), f32),
    )(u2e_r, u2e_t, csc)


def _padT_body1(a_ref, ao_ref):
    z = jnp.zeros((_PADR, 128 - DEG), jnp.int32)
    ao_ref[...] = jnp.concatenate([jnp.swapaxes(a_ref[...], 0, 1), z], axis=1)


def _pad_adj(*adjTs):
    i32 = jnp.int32
    n = adjTs[0].shape[1]
    grid = (n + _PADR - 1) // _PADR
    spec_in = pl.BlockSpec((DEG, _PADR), lambda i: (0, i))
    spec_out = pl.BlockSpec((_PADR, 128), lambda i: (i, 0))
    out = pl.pallas_call(
        _padT_body1,
        grid=(grid,),
        in_specs=[spec_in] * len(adjTs),
        out_specs=[spec_out] * len(adjTs),
        out_shape=[jax.ShapeDtypeStruct((n, 128), i32)] * len(adjTs),
    )(*adjTs)
    return out


def _mix_pad_body(r_ref, t_ref, c_ref, a_ref, b_ref, o_ref, ao_ref, bo_ref):
    _mix_body(r_ref, t_ref, c_ref, o_ref)
    z = jnp.zeros((_PADR, 128 - DEG), jnp.int32)
    ao_ref[...] = jnp.concatenate([jnp.swapaxes(a_ref[...], 0, 1), z], axis=1)
    bo_ref[...] = jnp.concatenate([jnp.swapaxes(b_ref[...], 0, 1), z], axis=1)


def _mix_and_pad(u2e_r, u2e_t, csc, u_uT, v_uT):
    f32 = jnp.float32
    i32 = jnp.int32
    n = u2e_r.shape[0]
    grid = (n + _MIXR - 1) // _MIXR
    return pl.pallas_call(
        _mix_pad_body,
        grid=(grid,),
        in_specs=[
            pl.BlockSpec((_MIXR, D), lambda i: (i, 0)),
            pl.BlockSpec((_MIXR, D), lambda i: (i, 0)),
            pl.BlockSpec(memory_space=pltpu.SMEM),
            pl.BlockSpec((DEG, _PADR), lambda i: (0, i)),
            pl.BlockSpec((DEG, _PADR), lambda i: (0, i)),
        ],
        out_specs=[
            pl.BlockSpec((_MIXR, D), lambda i: (i, 0)),
            pl.BlockSpec((_PADR, 128), lambda i: (i, 0)),
            pl.BlockSpec((_PADR, 128), lambda i: (i, 0)),
        ],
        out_shape=[
            jax.ShapeDtypeStruct((n, D), f32),
            jax.ShapeDtypeStruct((n, 128), i32),
            jax.ShapeDtypeStruct((n, 128), i32),
        ],
    )(u2e_r, u2e_t, csc, u_uT, v_uT)


def _bn(x, g, b):
    m = jnp.mean(x, axis=0, keepdims=True)
    v = jnp.mean((x - m) * (x - m), axis=0, keepdims=True)
    return g * (x - m) / jnp.sqrt(v + 1e-5) + b


def _tc_body(self_u, hist_agg, soc_agg, self_v, vagg,
             W_enc_u, b_enc_u, W_soc, b_soc, W_enc_v, b_enc_v,
             w_ur1, b_ur1, w_ur2, b_ur2, w_vr1, b_vr1, w_vr2, b_vr2,
             w_uv1, b_uv1, w_uv2, b_uv2, w_uv3, b_uv3,
             bn1_g, bn1_b, bn2_g, bn2_b, bn3_g, bn3_b, bn4_g, bn4_b,
             out):
    f32 = jnp.float32

    def mm(a, w):
        return lax.dot_general(a, w, (((1,), (0,)), ((), ())),
                               preferred_element_type=f32)

    hist = jnp.maximum(
        mm(jnp.concatenate([self_u[...], hist_agg[...]], axis=1),
           W_enc_u[...]) + b_enc_u[...], 0.0)
    emb_u = jnp.maximum(
        mm(jnp.concatenate([hist, soc_agg[...]], axis=1),
           W_soc[...]) + b_soc[...], 0.0)
    emb_v = jnp.maximum(
        mm(jnp.concatenate([self_v[...], vagg[...]], axis=1),
           W_enc_v[...]) + b_enc_v[...], 0.0)

    x_u = jnp.maximum(
        _bn(mm(emb_u, w_ur1[...]) + b_ur1[...], bn1_g[...], bn1_b[...]), 0.0)
    x_u = mm(x_u, w_ur2[...]) + b_ur2[...]
    x_v = jnp.maximum(
        _bn(mm(emb_v, w_vr1[...]) + b_vr1[...], bn2_g[...], bn2_b[...]), 0.0)
    x_v = mm(x_v, w_vr2[...]) + b_vr2[...]

    x = jnp.maximum(
        _bn(mm(jnp.concatenate([x_u, x_v], axis=1), w_uv1[...]) + b_uv1[...],
            bn3_g[...], bn3_b[...]), 0.0)
    x = jnp.maximum(
        _bn(mm(x, w_uv2[...]) + b_uv2[...], bn4_g[...], bn4_b[...]), 0.0)
    out[...] = (mm(x, w_uv3[...]) + b_uv3[...])[:, 0]


def kernel(params, nodes_u, nodes_v, labels_list, u_v, u_u, v_u):
    p = params
    f32 = jnp.float32
    i32 = jnp.int32
    nodes_u = nodes_u.astype(i32)
    nodes_v = nodes_v.astype(i32)

    bf16 = jnp.bfloat16
    csc = jnp.stack([p['share_p'][0, 0], p['share_p'][0, 1]]).astype(
        bf16).astype(f32).reshape(1, 2)

    coef = jnp.concatenate([
        jnp.full((16,), 1.0 / DEG, f32),
        jnp.zeros((112,), f32),
    ])

    (u_vp,) = _pad_adj(u_v.astype(i32).T)
    hist_agg, self_v = _sc_a(p['v2e_w'].astype(f32), nodes_u, nodes_v,
                             u_vp, coef)

    u2e_r, u2e_t, u_uT, v_uT, _ = lax.optimization_barrier(
        (p['u2e_r_w'].astype(f32), p['u2e_t_w'].astype(f32),
         u_u.astype(i32).T, v_u.astype(i32).T, u_vp))
    mixed, u_up, v_up = _mix_and_pad(u2e_r, u2e_t, csc, u_uT, v_uT)
    self_u, soc_agg, vagg = _sc_b(mixed, nodes_u, nodes_v, u_up, v_up, coef)

    def r2(v):
        return v.astype(f32).reshape(1, -1)

    scores = pl.pallas_call(
        _tc_body,
        out_shape=jax.ShapeDtypeStruct((B,), f32),
    )(self_u, hist_agg, soc_agg, self_v, vagg,
      p['W_enc_u'].astype(f32), r2(p['b_enc_u']),
      p['W_soc'].astype(f32), r2(p['b_soc']),
      p['W_enc_v'].astype(f32), r2(p['b_enc_v']),
      p['w_ur1'].astype(f32), r2(p['b_ur1']),
      p['w_ur2'].astype(f32), r2(p['b_ur2']),
      p['w_vr1'].astype(f32), r2(p['b_vr1']),
      p['w_vr2'].astype(f32), r2(p['b_vr2']),
      p['w_uv1'].astype(f32), r2(p['b_uv1']),
      p['w_uv2'].astype(f32), r2(p['b_uv2']),
      p['w_uv3'].astype(f32), r2(p['b_uv3']),
      r2(p['bn1_g']), r2(p['bn1_b']), r2(p['bn2_g']), r2(p['bn2_b']),
      r2(p['bn3_g']), r2(p['bn3_b']), r2(p['bn4_g']), r2(p['bn4_b']))
    return scores


# speedup vs baseline: 1.0133x; 1.0133x over previous
import jax
import jax.numpy as jnp
from jax import lax
from jax.experimental import pallas as pl
from jax.experimental.pallas import tpu as pltpu
from jax.experimental.pallas import tpu_sc as plsc

U = 50000
I = 50000
D = 128
B = 4096
DEG = 32

NC = 2
NS = 16
NW = NC * NS
BW = B // NW


def _transpose_adj(adj, adjT):
    lanes = lax.iota(jnp.int32, 16)

    def jbody(j, c):
        jv = jnp.full((16,), j, jnp.int32)
        for g in range(BW // 16):
            rows = lanes + (g * 16)
            vals = plsc.load_gather(adj, [rows, jv])
            adjT[j, pl.ds(g * 16, 16)] = vals
        return c

    lax.fori_loop(0, DEG, jbody, 0)


def _fire_mean_gather(table, adjT, acc, sem):
    pltpu.async_copy(table.at[adjT.at[0]], acc, sem).wait()

    def jbody(j, c):
        pltpu.async_copy(table.at[adjT.at[j]], acc, sem, add=True)
        return c

    lax.fori_loop(1, DEG, jbody, 0)


def _drain(table, acc, sem, count):
    def dbody(j, c):
        pltpu.make_async_copy(table.at[pl.ds(0, BW)], acc, sem).wait()
        return c

    lax.fori_loop(0, count, dbody, 0)


def _scale(accA, c0):
    def ibody(i, c):
        for g in range(D // 16):
            s = pl.ds(g * 16, 16)
            accA[i, s] = accA[i, s] * c0
        return c

    lax.fori_loop(0, BW, ibody, 0)


def _sc_a_body(v2e, nodes_u, nodes_v, u_v, coef,
               hist_o, self_v_o,
               nodes_b, nodes_b2, adj, adjT, accA, accB, coef_v, sem, sem2):
    wid = lax.axis_index("s") * NC + lax.axis_index("c")
    base = wid * BW

    pltpu.sync_copy(coef, coef_v)
    c_inv = coef_v[pl.ds(0, 16)]

    pltpu.sync_copy(nodes_u.at[pl.ds(base, BW)], nodes_b)
    adj_cp = pltpu.async_copy(u_v.at[nodes_b], adj, sem2)
    pltpu.sync_copy(nodes_v.at[pl.ds(base, BW)], nodes_b2)
    sv = pltpu.async_copy(v2e.at[nodes_b2], accB, sem)

    adj_cp.wait()
    _transpose_adj(adj, adjT)
    sv.wait()
    pltpu.sync_copy(accB, self_v_o.at[pl.ds(base, BW)])

    _fire_mean_gather(v2e, adjT, accA, sem)
    _drain(v2e, accA, sem, DEG - 1)
    _scale(accA, c_inv)
    pltpu.sync_copy(accA, hist_o.at[pl.ds(base, BW)])


def _sc_b_body(mixed, nodes_u, nodes_v, u_u, v_u, coef,
               self_u_o, soc_o, vagg_o,
               nodes_b, nodes_b2, adj, adj2, adjT, adjT2, accA, accB,
               coef_v, sem, sem2, sem3):
    wid = lax.axis_index("s") * NC + lax.axis_index("c")
    base = wid * BW

    pltpu.sync_copy(coef, coef_v)
    c_inv = coef_v[pl.ds(0, 16)]

    pltpu.sync_copy(nodes_u.at[pl.ds(base, BW)], nodes_b)
    adj_cp = pltpu.async_copy(u_u.at[nodes_b], adj, sem2)
    su = pltpu.async_copy(mixed.at[nodes_b], accB, sem)
    pltpu.sync_copy(nodes_v.at[pl.ds(base, BW)], nodes_b2)
    adj_cp2 = pltpu.async_copy(v_u.at[nodes_b2], adj2, sem2)

    adj_cp.wait()
    _transpose_adj(adj, adjT)
    su.wait()
    pltpu.sync_copy(accB, self_u_o.at[pl.ds(base, BW)])

    _fire_mean_gather(mixed, adjT, accA, sem)
    adj_cp2.wait()
    _transpose_adj(adj2, adjT2)
    _fire_mean_gather(mixed, adjT2, accB, sem3)
    _drain(mixed, accA, sem, DEG - 1)
    _scale(accA, c_inv)
    pltpu.sync_copy(accA, soc_o.at[pl.ds(base, BW)])
    _drain(mixed, accB, sem3, DEG - 1)
    _scale(accB, c_inv)
    pltpu.sync_copy(accB, vagg_o.at[pl.ds(base, BW)])


def _sc_mesh():
    return plsc.VectorSubcoreMesh(
        core_axis_name="c", subcore_axis_name="s",
        num_cores=NC, num_subcores=NS)


def _sc_a(v2e, nodes_u, nodes_v, u_vp, coef):
    f32 = jnp.float32
    i32 = jnp.int32
    run = pl.kernel(
        _sc_a_body,
        out_type=tuple(jax.ShapeDtypeStruct((B, D), f32) for _ in range(2)),
        mesh=_sc_mesh(),
        compiler_params=pltpu.CompilerParams(needs_layout_passes=False),
        scratch_types=[
            pltpu.VMEM((BW,), i32),
            pltpu.VMEM((BW,), i32),
            pltpu.VMEM((BW, 128), i32),
            pltpu.VMEM((DEG, BW), i32),
            pltpu.VMEM((BW, D), f32),
            pltpu.VMEM((BW, D), f32),
            pltpu.VMEM((128,), f32),
            pltpu.SemaphoreType.DMA,
            pltpu.SemaphoreType.DMA,
        ],
    )
    return run(v2e, nodes_u, nodes_v, u_vp, coef)


def _sc_b(mixed, nodes_u, nodes_v, u_up, v_up, coef):
    f32 = jnp.float32
    i32 = jnp.int32
    run = pl.kernel(
        _sc_b_body,
        out_type=tuple(jax.ShapeDtypeStruct((B, D), f32) for _ in range(3)),
        mesh=_sc_mesh(),
        compiler_params=pltpu.CompilerParams(needs_layout_passes=False),
        scratch_types=[
            pltpu.VMEM((BW,), i32),
            pltpu.VMEM((BW,), i32),
            pltpu.VMEM((BW, 128), i32),
            pltpu.VMEM((BW, 128), i32),
            pltpu.VMEM((DEG, BW), i32),
            pltpu.VMEM((DEG, BW), i32),
            pltpu.VMEM((BW, D), f32),
            pltpu.VMEM((BW, D), f32),
            pltpu.VMEM((128,), f32),
            pltpu.SemaphoreType.DMA,
            pltpu.SemaphoreType.DMA,
            pltpu.SemaphoreType.DMA,
        ],
    )
    return run(mixed, nodes_u, nodes_v, u_up, v_up, coef)


_MIXR = 4096
_PADR = 4096


def _mix_body(r_ref, t_ref, c_ref, o_ref):
    bf16 = jnp.bfloat16
    f32 = jnp.float32
    r = r_ref[...].astype(bf16).astype(f32)
    t = t_ref[...].astype(bf16).astype(f32)
    o_ref[...] = c_ref[0, 0] * r + c_ref[0, 1] * t


def _mix_table(u2e_r, u2e_t, csc):
    f32 = jnp.float32
    n = u2e_r.shape[0]
    grid = (n + _MIXR - 1) // _MIXR
    return pl.pallas_call(
        _mix_body,
        grid=(grid,),
        in_specs=[
            pl.BlockSpec((_MIXR, D), lambda i: (i, 0)),
            pl.BlockSpec((_MIXR, D), lambda i: (i, 0)),
            pl.BlockSpec(memory_space=pltpu.SMEM),
        ],
        out_specs=pl.BlockSpec((_MIXR, D), lambda i: (i, 0)),
        out_shape=jax.ShapeDtypeStruct((n, D), f32),
    )(u2e_r, u2e_t, csc)


def _padT_body1(a_ref, ao_ref):
    z = jnp.zeros((_PADR, 128 - DEG), jnp.int32)
    ao_ref[...] = jnp.concatenate([jnp.swapaxes(a_ref[...], 0, 1), z], axis=1)


def _padT_body2(a_ref, b_ref, ao_ref, bo_ref):
    z = jnp.zeros((_PADR, 128 - DEG), jnp.int32)
    ao_ref[...] = jnp.concatenate([jnp.swapaxes(a_ref[...], 0, 1), z], axis=1)
    bo_ref[...] = jnp.concatenate([jnp.swapaxes(b_ref[...], 0, 1), z], axis=1)


def _pad_adj(*adjTs):
    i32 = jnp.int32
    n = adjTs[0].shape[1]
    grid = (n + _PADR - 1) // _PADR
    spec_in = pl.BlockSpec((DEG, _PADR), lambda i: (0, i))
    spec_out = pl.BlockSpec((_PADR, 128), lambda i: (i, 0))
    out = pl.pallas_call(
        _padT_body1 if len(adjTs) == 1 else _padT_body2,
        grid=(grid,),
        in_specs=[spec_in] * len(adjTs),
        out_specs=[spec_out] * len(adjTs),
        out_shape=[jax.ShapeDtypeStruct((n, 128), i32)] * len(adjTs),
    )(*adjTs)
    return out


def _mix_pad_body(r_ref, t_ref, c_ref, a_ref, b_ref, o_ref, ao_ref, bo_ref):
    _mix_body(r_ref, t_ref, c_ref, o_ref)
    z = jnp.zeros((_PADR, 128 - DEG), jnp.int32)
    ao_ref[...] = jnp.concatenate([jnp.swapaxes(a_ref[...], 0, 1), z], axis=1)
    bo_ref[...] = jnp.concatenate([jnp.swapaxes(b_ref[...], 0, 1), z], axis=1)


def _mix_and_pad(u2e_r, u2e_t, csc, u_uT, v_uT):
    f32 = jnp.float32
    i32 = jnp.int32
    n = u2e_r.shape[0]
    grid = (n + _MIXR - 1) // _MIXR
    return pl.pallas_call(
        _mix_pad_body,
        grid=(grid,),
        in_specs=[
            pl.BlockSpec((_MIXR, D), lambda i: (i, 0)),
            pl.BlockSpec((_MIXR, D), lambda i: (i, 0)),
            pl.BlockSpec(memory_space=pltpu.SMEM),
            pl.BlockSpec((DEG, _PADR), lambda i: (0, i)),
            pl.BlockSpec((DEG, _PADR), lambda i: (0, i)),
        ],
        out_specs=[
            pl.BlockSpec((_MIXR, D), lambda i: (i, 0)),
            pl.BlockSpec((_PADR, 128), lambda i: (i, 0)),
            pl.BlockSpec((_PADR, 128), lambda i: (i, 0)),
        ],
        out_shape=[
            jax.ShapeDtypeStruct((n, D), f32),
            jax.ShapeDtypeStruct((n, 128), i32),
            jax.ShapeDtypeStruct((n, 128), i32),
        ],
    )(u2e_r, u2e_t, csc, u_uT, v_uT)


def _bn(x, g, b):
    m = jnp.mean(x, axis=0, keepdims=True)
    v = jnp.mean((x - m) * (x - m), axis=0, keepdims=True)
    return g * (x - m) / jnp.sqrt(v + 1e-5) + b


def _tc_body(self_u, hist_agg, soc_agg, self_v, vagg,
             W_enc_u, b_enc_u, W_soc, b_soc, W_enc_v, b_enc_v,
             w_ur1, b_ur1, w_ur2, b_ur2, w_vr1, b_vr1, w_vr2, b_vr2,
             w_uv1, b_uv1, w_uv2, b_uv2, w_uv3, b_uv3,
             bn1_g, bn1_b, bn2_g, bn2_b, bn3_g, bn3_b, bn4_g, bn4_b,
             out):
    f32 = jnp.float32

    def mm(a, w):
        return lax.dot_general(a, w, (((1,), (0,)), ((), ())),
                               preferred_element_type=f32)

    hist = jnp.maximum(
        mm(jnp.concatenate([self_u[...], hist_agg[...]], axis=1),
           W_enc_u[...]) + b_enc_u[...], 0.0)
    emb_u = jnp.maximum(
        mm(jnp.concatenate([hist, soc_agg[...]], axis=1),
           W_soc[...]) + b_soc[...], 0.0)
    emb_v = jnp.maximum(
        mm(jnp.concatenate([self_v[...], vagg[...]], axis=1),
           W_enc_v[...]) + b_enc_v[...], 0.0)

    x_u = jnp.maximum(
        _bn(mm(emb_u, w_ur1[...]) + b_ur1[...], bn1_g[...], bn1_b[...]), 0.0)
    x_u = mm(x_u, w_ur2[...]) + b_ur2[...]
    x_v = jnp.maximum(
        _bn(mm(emb_v, w_vr1[...]) + b_vr1[...], bn2_g[...], bn2_b[...]), 0.0)
    x_v = mm(x_v, w_vr2[...]) + b_vr2[...]

    x = jnp.maximum(
        _bn(mm(jnp.concatenate([x_u, x_v], axis=1), w_uv1[...]) + b_uv1[...],
            bn3_g[...], bn3_b[...]), 0.0)
    x = jnp.maximum(
        _bn(mm(x, w_uv2[...]) + b_uv2[...], bn4_g[...], bn4_b[...]), 0.0)
    out[...] = (mm(x, w_uv3[...]) + b_uv3[...])[:, 0]


def kernel(params, nodes_u, nodes_v, labels_list, u_v, u_u, v_u):
    p = params
    f32 = jnp.float32
    i32 = jnp.int32
    nodes_u = nodes_u.astype(i32)
    nodes_v = nodes_v.astype(i32)

    bf16 = jnp.bfloat16
    csc = jnp.stack([p['share_p'][0, 0], p['share_p'][0, 1]]).astype(
        bf16).astype(f32).reshape(1, 2)

    coef = jnp.concatenate([
        jnp.full((16,), 1.0 / DEG, f32),
        jnp.zeros((112,), f32),
    ])

    (u_vp,) = _pad_adj(u_v.astype(i32).T)
    hist_agg, self_v = _sc_a(p['v2e_w'].astype(f32), nodes_u, nodes_v,
                             u_vp, coef)

    u2e_r, u2e_t, u_uT, v_uT, _ = lax.optimization_barrier(
        (p['u2e_r_w'].astype(f32), p['u2e_t_w'].astype(f32),
         u_u.astype(i32).T, v_u.astype(i32).T, u_vp))
    mixed = _mix_table(u2e_r, u2e_t, csc)
    u_up, v_up = _pad_adj(u_uT, v_uT)
    self_u, soc_agg, vagg = _sc_b(mixed, nodes_u, nodes_v, u_up, v_up, coef)

    def r2(v):
        return v.astype(f32).reshape(1, -1)

    scores = pl.pallas_call(
        _tc_body,
        out_shape=jax.ShapeDtypeStruct((B,), f32),
    )(self_u, hist_agg, soc_agg, self_v, vagg,
      p['W_enc_u'].astype(f32), r2(p['b_enc_u']),
      p['W_soc'].astype(f32), r2(p['b_soc']),
      p['W_enc_v'].astype(f32), r2(p['b_enc_v']),
      p['w_ur1'].astype(f32), r2(p['b_ur1']),
      p['w_ur2'].astype(f32), r2(p['b_ur2']),
      p['w_vr1'].astype(f32), r2(p['b_vr1']),
      p['w_vr2'].astype(f32), r2(p['b_vr2']),
      p['w_uv1'].astype(f32), r2(p['b_uv1']),
      p['w_uv2'].astype(f32), r2(p['b_uv2']),
      p['w_uv3'].astype(f32), r2(p['b_uv3']),
      r2(p['bn1_g']), r2(p['bn1_b']), r2(p['bn2_g']), r2(p['bn2_b']),
      r2(p['bn3_g']), r2(p['bn3_b']), r2(p['bn4_g']), r2(p['bn4_b']))
    return scores


# speedup vs baseline: 1.0135x; 1.0002x over previous
import jax
import jax.numpy as jnp
from jax import lax
from jax.experimental import pallas as pl
from jax.experimental.pallas import tpu as pltpu
from jax.experimental.pallas import tpu_sc as plsc

U = 50000
I = 50000
D = 128
B = 4096
DEG = 32

NC = 2
NS = 16
NW = NC * NS
BW = B // NW


def _transpose_adj(adj, adjT):
    lanes = lax.iota(jnp.int32, 16)

    def jbody(j, c):
        jv = jnp.full((16,), j, jnp.int32)
        for g in range(BW // 16):
            rows = lanes + (g * 16)
            vals = plsc.load_gather(adj, [rows, jv])
            adjT[j, pl.ds(g * 16, 16)] = vals
        return c

    lax.fori_loop(0, DEG, jbody, 0)


def _fire_mean_gather(table, adjT, acc, sem):
    pltpu.async_copy(table.at[adjT.at[0]], acc, sem).wait()

    def jbody(j, c):
        pltpu.async_copy(table.at[adjT.at[j]], acc, sem, add=True)
        return c

    lax.fori_loop(1, DEG, jbody, 0)


def _drain(table, acc, sem, count):
    def dbody(j, c):
        pltpu.make_async_copy(table.at[pl.ds(0, BW)], acc, sem).wait()
        return c

    lax.fori_loop(0, count, dbody, 0)


def _scale(accA, c0):
    def ibody(i, c):
        for g in range(D // 16):
            s = pl.ds(g * 16, 16)
            accA[i, s] = accA[i, s] * c0
        return c

    lax.fori_loop(0, BW, ibody, 0)


def _sc_a_body(v2e, nodes_u, nodes_v, u_v, coef,
               hist_o, self_v_o,
               nodes_b, nodes_b2, adj, adjT, accA, accB, coef_v, sem, sem2):
    wid = lax.axis_index("s") * NC + lax.axis_index("c")
    base = wid * BW

    pltpu.sync_copy(coef, coef_v)
    c_inv = coef_v[pl.ds(0, 16)]

    pltpu.sync_copy(nodes_u.at[pl.ds(base, BW)], nodes_b)
    adj_cp = pltpu.async_copy(u_v.at[nodes_b], adj, sem2)
    pltpu.sync_copy(nodes_v.at[pl.ds(base, BW)], nodes_b2)
    sv = pltpu.async_copy(v2e.at[nodes_b2], accB, sem)

    adj_cp.wait()
    _transpose_adj(adj, adjT)
    sv.wait()
    pltpu.sync_copy(accB, self_v_o.at[pl.ds(base, BW)])

    _fire_mean_gather(v2e, adjT, accA, sem)
    _drain(v2e, accA, sem, DEG - 1)
    _scale(accA, c_inv)
    pltpu.sync_copy(accA, hist_o.at[pl.ds(base, BW)])


def _sc_b_body(mixed, nodes_u, nodes_v, u_u, v_u, coef,
               self_u_o, soc_o, vagg_o,
               nodes_b, nodes_b2, adj, adj2, adjT, adjT2, accA, accB,
               coef_v, sem, sem2, sem3):
    wid = lax.axis_index("s") * NC + lax.axis_index("c")
    base = wid * BW

    pltpu.sync_copy(coef, coef_v)
    c_inv = coef_v[pl.ds(0, 16)]

    pltpu.sync_copy(nodes_u.at[pl.ds(base, BW)], nodes_b)
    adj_cp = pltpu.async_copy(u_u.at[nodes_b], adj, sem2)
    su = pltpu.async_copy(mixed.at[nodes_b], accB, sem)
    pltpu.sync_copy(nodes_v.at[pl.ds(base, BW)], nodes_b2)
    adj_cp2 = pltpu.async_copy(v_u.at[nodes_b2], adj2, sem2)

    adj_cp.wait()
    _transpose_adj(adj, adjT)
    su.wait()
    pltpu.sync_copy(accB, self_u_o.at[pl.ds(base, BW)])

    _fire_mean_gather(mixed, adjT, accA, sem)
    adj_cp2.wait()
    _transpose_adj(adj2, adjT2)
    _fire_mean_gather(mixed, adjT2, accB, sem3)
    _drain(mixed, accA, sem, DEG - 1)
    _scale(accA, c_inv)
    pltpu.sync_copy(accA, soc_o.at[pl.ds(base, BW)])
    _drain(mixed, accB, sem3, DEG - 1)
    _scale(accB, c_inv)
    pltpu.sync_copy(accB, vagg_o.at[pl.ds(base, BW)])


def _sc_mesh():
    return plsc.VectorSubcoreMesh(
        core_axis_name="c", subcore_axis_name="s",
        num_cores=NC, num_subcores=NS)


def _sc_a(v2e, nodes_u, nodes_v, u_vp, coef):
    f32 = jnp.float32
    i32 = jnp.int32
    run = pl.kernel(
        _sc_a_body,
        out_type=tuple(jax.ShapeDtypeStruct((B, D), f32) for _ in range(2)),
        mesh=_sc_mesh(),
        compiler_params=pltpu.CompilerParams(needs_layout_passes=False),
        scratch_types=[
            pltpu.VMEM((BW,), i32),
            pltpu.VMEM((BW,), i32),
            pltpu.VMEM((BW, 128), i32),
            pltpu.VMEM((DEG, BW), i32),
            pltpu.VMEM((BW, D), f32),
            pltpu.VMEM((BW, D), f32),
            pltpu.VMEM((128,), f32),
            pltpu.SemaphoreType.DMA,
            pltpu.SemaphoreType.DMA,
        ],
    )
    return run(v2e, nodes_u, nodes_v, u_vp, coef)


def _sc_b(mixed, nodes_u, nodes_v, u_up, v_up, coef):
    f32 = jnp.float32
    i32 = jnp.int32
    run = pl.kernel(
        _sc_b_body,
        out_type=tuple(jax.ShapeDtypeStruct((B, D), f32) for _ in range(3)),
        mesh=_sc_mesh(),
        compiler_params=pltpu.CompilerParams(needs_layout_passes=False),
        scratch_types=[
            pltpu.VMEM((BW,), i32),
            pltpu.VMEM((BW,), i32),
            pltpu.VMEM((BW, 128), i32),
            pltpu.VMEM((BW, 128), i32),
            pltpu.VMEM((DEG, BW), i32),
            pltpu.VMEM((DEG, BW), i32),
            pltpu.VMEM((BW, D), f32),
            pltpu.VMEM((BW, D), f32),
            pltpu.VMEM((128,), f32),
            pltpu.SemaphoreType.DMA,
            pltpu.SemaphoreType.DMA,
            pltpu.SemaphoreType.DMA,
        ],
    )
    return run(mixed, nodes_u, nodes_v, u_up, v_up, coef)


_MIXR = 4096
_PADR = 4096


def _mix_body(r_ref, t_ref, c_ref, o_ref):
    bf16 = jnp.bfloat16
    f32 = jnp.float32
    r = r_ref[...].astype(bf16).astype(f32)
    t = t_ref[...].astype(bf16).astype(f32)
    o_ref[...] = c_ref[0, 0] * r + c_ref[0, 1] * t


def _mix_table(u2e_r, u2e_t, csc):
    f32 = jnp.float32
    n = u2e_r.shape[0]
    grid = (n + _MIXR - 1) // _MIXR
    return pl.pallas_call(
        _mix_body,
        grid=(grid,),
        in_specs=[
            pl.BlockSpec((_MIXR, D), lambda i: (i, 0)),
            pl.BlockSpec((_MIXR, D), lambda i: (i, 0)),
            pl.BlockSpec(memory_space=pltpu.SMEM),
        ],
        out_specs=pl.BlockSpec((_MIXR, D), lambda i: (i, 0)),
        out_shape=jax.ShapeDtypeStruct((n, D), f32),
    )(u2e_r, u2e_t, csc)


def _padT_body1(a_ref, ao_ref):
    z = jnp.zeros((_PADR, 128 - DEG), jnp.int32)
    ao_ref[...] = jnp.concatenate([jnp.swapaxes(a_ref[...], 0, 1), z], axis=1)


def _padT_body2(a_ref, b_ref, ao_ref, bo_ref):
    z = jnp.zeros((_PADR, 128 - DEG), jnp.int32)
    ao_ref[...] = jnp.concatenate([jnp.swapaxes(a_ref[...], 0, 1), z], axis=1)
    bo_ref[...] = jnp.concatenate([jnp.swapaxes(b_ref[...], 0, 1), z], axis=1)


def _pad_adj(*adjTs):
    i32 = jnp.int32
    n = adjTs[0].shape[1]
    grid = (n + _PADR - 1) // _PADR
    spec_in = pl.BlockSpec((DEG, _PADR), lambda i: (0, i))
    spec_out = pl.BlockSpec((_PADR, 128), lambda i: (i, 0))
    out = pl.pallas_call(
        _padT_body1 if len(adjTs) == 1 else _padT_body2,
        grid=(grid,),
        in_specs=[spec_in] * len(adjTs),
        out_specs=[spec_out] * len(adjTs),
        out_shape=[jax.ShapeDtypeStruct((n, 128), i32)] * len(adjTs),
    )(*adjTs)
    return out


def _bn(x, g, b):
    m = jnp.mean(x, axis=0, keepdims=True)
    v = jnp.mean((x - m) * (x - m), axis=0, keepdims=True)
    return g * (x - m) / jnp.sqrt(v + 1e-5) + b


def _tc_body(self_u, hist_agg, soc_agg, self_v, vagg,
             W_enc_u, b_enc_u, W_soc, b_soc, W_enc_v, b_enc_v,
             w_ur1, b_ur1, w_ur2, b_ur2, w_vr1, b_vr1, w_vr2, b_vr2,
             w_uv1, b_uv1, w_uv2, b_uv2, w_uv3, b_uv3,
             bn1_g, bn1_b, bn2_g, bn2_b, bn3_g, bn3_b, bn4_g, bn4_b,
             out):
    f32 = jnp.float32

    def mm(a, w):
        return lax.dot_general(a, w, (((1,), (0,)), ((), ())),
                               preferred_element_type=f32)

    hist = jnp.maximum(
        mm(jnp.concatenate([self_u[...], hist_agg[...]], axis=1),
           W_enc_u[...]) + b_enc_u[...], 0.0)
    emb_u = jnp.maximum(
        mm(jnp.concatenate([hist, soc_agg[...]], axis=1),
           W_soc[...]) + b_soc[...], 0.0)
    emb_v = jnp.maximum(
        mm(jnp.concatenate([self_v[...], vagg[...]], axis=1),
           W_enc_v[...]) + b_enc_v[...], 0.0)

    x_u = jnp.maximum(
        _bn(mm(emb_u, w_ur1[...]) + b_ur1[...], bn1_g[...], bn1_b[...]), 0.0)
    x_u = mm(x_u, w_ur2[...]) + b_ur2[...]
    x_v = jnp.maximum(
        _bn(mm(emb_v, w_vr1[...]) + b_vr1[...], bn2_g[...], bn2_b[...]), 0.0)
    x_v = mm(x_v, w_vr2[...]) + b_vr2[...]

    x = jnp.maximum(
        _bn(mm(jnp.concatenate([x_u, x_v], axis=1), w_uv1[...]) + b_uv1[...],
            bn3_g[...], bn3_b[...]), 0.0)
    x = jnp.maximum(
        _bn(mm(x, w_uv2[...]) + b_uv2[...], bn4_g[...], bn4_b[...]), 0.0)
    out[...] = (mm(x, w_uv3[...]) + b_uv3[...])[:, 0]


def kernel(params, nodes_u, nodes_v, labels_list, u_v, u_u, v_u):
    p = params
    f32 = jnp.float32
    i32 = jnp.int32
    nodes_u = nodes_u.astype(i32)
    nodes_v = nodes_v.astype(i32)

    bf16 = jnp.bfloat16
    csc = jnp.stack([p['share_p'][0, 0], p['share_p'][0, 1]]).astype(
        bf16).astype(f32).reshape(1, 2)

    coef = jnp.concatenate([
        jnp.full((16,), 1.0 / DEG, f32),
        jnp.zeros((112,), f32),
    ])

    (u_vp,) = _pad_adj(u_v.astype(i32).T)
    hist_agg, self_v = _sc_a(p['v2e_w'].astype(f32), nodes_u, nodes_v,
                             u_vp, coef)

    u2e_r, u2e_t, u_uT, v_uT, _ = lax.optimization_barrier(
        (p['u2e_r_w'].astype(f32), p['u2e_t_w'].astype(f32),
         u_u.astype(i32).T, v_u.astype(i32).T, u_vp))
    mixed = _mix_table(u2e_r, u2e_t, csc)
    u_up, v_up = _pad_adj(u_uT, v_uT)
    self_u, soc_agg, vagg = _sc_b(mixed, nodes_u, nodes_v, u_up, v_up, coef)

    def r2(v):
        return v.astype(f32).reshape(1, -1)

    scores = pl.pallas_call(
        _tc_body,
        out_shape=jax.ShapeDtypeStruct((B,), f32),
    )(self_u, hist_agg, soc_agg, self_v, vagg,
      p['W_enc_u'].astype(f32), r2(p['b_enc_u']),
      p['W_soc'].astype(f32), r2(p['b_soc']),
      p['W_enc_v'].astype(f32), r2(p['b_enc_v']),
      p['w_ur1'].astype(f32), r2(p['b_ur1']),
      p['w_ur2'].astype(f32), r2(p['b_ur2']),
      p['w_vr1'].astype(f32), r2(p['b_vr1']),
      p['w_vr2'].astype(f32), r2(p['b_vr2']),
      p['w_uv1'].astype(f32), r2(p['b_uv1']),
      p['w_uv2'].astype(f32), r2(p['b_uv2']),
      p['w_uv3'].astype(f32), r2(p['b_uv3']),
      r2(p['bn1_g']), r2(p['bn1_b']), r2(p['bn2_g']), r2(p['bn2_b']),
      r2(p['bn3_g']), r2(p['bn3_b']), r2(p['bn4_g']), r2(p['bn4_b']))
    return scores
